# Initial kernel scaffold; baseline (speedup 1.0000x reference)
import jax
import jax.numpy as jnp
from jax.experimental import pallas as pl


def kernel(sentences_indices, embedding_table):
    raise NotImplementedError("write your pallas kernel here")


# speedup vs baseline: 7.4368x; 7.4368x over previous
import jax
import jax.numpy as jnp
from jax.experimental import pallas as pl
from jax.experimental.pallas import tpu as pltpu
from jax.experimental.pallas import tpu_sc as plsc

EMBEDDING_DIM = 128
WINDOW = 128


def kernel(sentences_indices, embedding_table):
    batch, hist = sentences_indices.shape
    num_indices = batch * hist
    flat_idx = sentences_indices.reshape(1, num_indices).astype(jnp.int32)

    mesh = plsc.VectorSubcoreMesh(
        core_axis_name="core", subcore_axis_name="subcore"
    )

    @pl.kernel(
        out_type=jax.ShapeDtypeStruct((num_indices, EMBEDDING_DIM), jnp.float32),
        mesh=mesh,
    )
    def gather_kernel(table_hbm, idx_hbm, out_hbm):
        def body(idx_vmem, out_vmem):
            pltpu.sync_copy(table_hbm.at[idx_vmem.at[0]], out_vmem)

        pltpu.emit_pipeline(
            body,
            grid=(num_indices // WINDOW,),
            in_specs=[
                pl.BlockSpec((1, WINDOW), index_map=lambda i: (0, i))
            ],
            out_specs=[
                pl.BlockSpec((WINDOW, EMBEDDING_DIM), index_map=lambda i: (i, 0))
            ],
            core_axis_name=("core", "subcore"),
            dimension_semantics=(pltpu.PARALLEL,),
        )(idx_hbm, out_hbm)

    out = gather_kernel(embedding_table, flat_idx)
    return out.reshape(batch, hist, EMBEDDING_DIM)


# speedup vs baseline: 9.2115x; 1.2386x over previous
import jax
import jax.numpy as jnp
from jax import lax
from jax.experimental import pallas as pl
from jax.experimental.pallas import tpu as pltpu
from jax.experimental.pallas import tpu_sc as plsc

EMBEDDING_DIM = 128
WINDOW = 128
NBUF = 4
NUM_CORES = 2
NUM_SUBCORES = 16
NUM_WORKERS = NUM_CORES * NUM_SUBCORES


def kernel(sentences_indices, embedding_table):
    batch, hist = sentences_indices.shape
    num_indices = batch * hist
    steps_per_worker = num_indices // (NUM_WORKERS * WINDOW)
    idx2d = sentences_indices.reshape(num_indices // WINDOW, WINDOW).astype(
        jnp.int32
    )

    mesh = plsc.VectorSubcoreMesh(
        core_axis_name="core", subcore_axis_name="subcore"
    )

    @pl.kernel(
        out_type=jax.ShapeDtypeStruct((num_indices, EMBEDDING_DIM), jnp.float32),
        mesh=mesh,
        scratch_types=[
            pltpu.VMEM((steps_per_worker, WINDOW), jnp.int32),
            pltpu.VMEM((NBUF, WINDOW, EMBEDDING_DIM), jnp.float32),
            pltpu.SemaphoreType.DMA((NBUF,)),
            pltpu.SemaphoreType.DMA((NBUF,)),
        ],
    )
    def gather_kernel(table_hbm, idx_hbm, out_hbm, idx_v, bufs, gsem, osem):
        wid = lax.axis_index("subcore") * NUM_CORES + lax.axis_index("core")
        row0 = wid * steps_per_worker
        base = row0 * WINDOW

        pltpu.sync_copy(idx_hbm.at[pl.ds(row0, steps_per_worker)], idx_v)

        for b in range(NBUF):
            pltpu.async_copy(table_hbm.at[idx_v.at[b]], bufs.at[b], gsem.at[b])

        @pl.loop(0, steps_per_worker - NBUF, step=NBUF)
        def _(jo):
            for b in range(NBUF):
                j = jo + b
                pltpu.make_async_copy(
                    table_hbm.at[idx_v.at[j]], bufs.at[b], gsem.at[b]
                ).wait()
                pltpu.async_copy(
                    bufs.at[b],
                    out_hbm.at[pl.ds(base + j * WINDOW, WINDOW)],
                    osem.at[b],
                )
                pltpu.make_async_copy(
                    bufs.at[b],
                    out_hbm.at[pl.ds(base + j * WINDOW, WINDOW)],
                    osem.at[b],
                ).wait()
                pltpu.async_copy(
                    table_hbm.at[idx_v.at[j + NBUF]], bufs.at[b], gsem.at[b]
                )

        for b in range(NBUF):
            j = steps_per_worker - NBUF + b
            pltpu.make_async_copy(
                table_hbm.at[idx_v.at[j]], bufs.at[b], gsem.at[b]
            ).wait()
            pltpu.async_copy(
                bufs.at[b],
                out_hbm.at[pl.ds(base + j * WINDOW, WINDOW)],
                osem.at[b],
            )
        for b in range(NBUF):
            j = steps_per_worker - NBUF + b
            pltpu.make_async_copy(
                bufs.at[b],
                out_hbm.at[pl.ds(base + j * WINDOW, WINDOW)],
                osem.at[b],
            ).wait()

    out = gather_kernel(embedding_table, idx2d)
    return out.reshape(batch, hist, EMBEDDING_DIM)


# speedup vs baseline: 9.2260x; 1.0016x over previous
import jax
import jax.numpy as jnp
from jax import lax
from jax.experimental import pallas as pl
from jax.experimental.pallas import tpu as pltpu
from jax.experimental.pallas import tpu_sc as plsc

EMBEDDING_DIM = 128
WINDOW = 128
NBUF = 5
NUM_CORES = 2
NUM_SUBCORES = 16
NUM_WORKERS = NUM_CORES * NUM_SUBCORES


def kernel(sentences_indices, embedding_table):
    batch, hist = sentences_indices.shape
    num_indices = batch * hist
    steps_per_worker = num_indices // (NUM_WORKERS * WINDOW)
    idx2d = sentences_indices.reshape(num_indices // WINDOW, WINDOW).astype(
        jnp.int32
    )

    mesh = plsc.VectorSubcoreMesh(
        core_axis_name="core", subcore_axis_name="subcore"
    )

    @pl.kernel(
        out_type=jax.ShapeDtypeStruct((num_indices, EMBEDDING_DIM), jnp.float32),
        mesh=mesh,
        scratch_types=[
            pltpu.VMEM((steps_per_worker, WINDOW), jnp.int32),
            pltpu.VMEM((NBUF, WINDOW, EMBEDDING_DIM), jnp.float32),
            pltpu.SemaphoreType.DMA((NBUF,)),
            pltpu.SemaphoreType.DMA((NBUF,)),
        ],
    )
    def gather_kernel(table_hbm, idx_hbm, out_hbm, idx_v, bufs, gsem, osem):
        wid = lax.axis_index("subcore") * NUM_CORES + lax.axis_index("core")
        row0 = wid * steps_per_worker
        base = row0 * WINDOW

        pltpu.sync_copy(idx_hbm.at[pl.ds(row0, steps_per_worker)], idx_v)

        for b in range(NBUF):
            pltpu.async_copy(table_hbm.at[idx_v.at[b]], bufs.at[b], gsem.at[b])

        @pl.loop(0, steps_per_worker - NBUF, step=NBUF)
        def _(jo):
            for b in range(NBUF):
                j = jo + b
                pltpu.make_async_copy(
                    table_hbm.at[idx_v.at[j]], bufs.at[b], gsem.at[b]
                ).wait()
                pltpu.async_copy(
                    bufs.at[b],
                    out_hbm.at[pl.ds(base + j * WINDOW, WINDOW)],
                    osem.at[b],
                )
                pltpu.make_async_copy(
                    bufs.at[b],
                    out_hbm.at[pl.ds(base + j * WINDOW, WINDOW)],
                    osem.at[b],
                ).wait()
                pltpu.async_copy(
                    table_hbm.at[idx_v.at[j + NBUF]], bufs.at[b], gsem.at[b]
                )

        for b in range(NBUF):
            j = steps_per_worker - NBUF + b
            pltpu.make_async_copy(
                table_hbm.at[idx_v.at[j]], bufs.at[b], gsem.at[b]
            ).wait()
            pltpu.async_copy(
                bufs.at[b],
                out_hbm.at[pl.ds(base + j * WINDOW, WINDOW)],
                osem.at[b],
            )
        for b in range(NBUF):
            j = steps_per_worker - NBUF + b
            pltpu.make_async_copy(
                bufs.at[b],
                out_hbm.at[pl.ds(base + j * WINDOW, WINDOW)],
                osem.at[b],
            ).wait()

    out = gather_kernel(embedding_table, idx2d)
    return out.reshape(batch, hist, EMBEDDING_DIM)
